# baseline (speedup 1.0000x reference)
import functools

import jax
import jax.numpy as jnp
from jax import lax
from jax.experimental import pallas as pl
from jax.experimental.pallas import tpu as pltpu
from jax.experimental.pallas import tpu_sc as plsc

R = 64
V = 1_000_000
B = 16384
NC = 2
NS = 16
NW = NC * NS
ROWS_PER_W = R // NW
L = 16


def _sc_gather(t_flat, idx32):
    mesh = plsc.VectorSubcoreMesh(core_axis_name="c", subcore_axis_name="s")

    @functools.partial(
        pl.kernel,
        mesh=mesh,
        out_type=jax.ShapeDtypeStruct((R, B), jnp.float32),
        scratch_types=[
            pltpu.VMEM((B,), jnp.int32),
            pltpu.VMEM((B,), jnp.int32),
            pltpu.VMEM((B,), jnp.float32),
            pltpu.SemaphoreType.DMA,
        ],
    )
    def k(t_hbm, idx_hbm, out_hbm, idx_v, absidx_v, row_v, sem):
        wid = lax.axis_index("s") * NC + lax.axis_index("c")
        pltpu.sync_copy(idx_hbm, idx_v)
        for r_local in range(ROWS_PER_W):
            row = wid * ROWS_PER_W + r_local
            base = row * V

            def body(i, _):
                sl = pl.ds(i * L, L)
                absidx_v[sl] = idx_v[sl] + base
                return 0

            lax.fori_loop(0, B // L, body, 0)
            pltpu.async_copy(t_hbm.at[absidx_v], row_v, sem).wait()
            pltpu.sync_copy(row_v, out_hbm.at[row])

    return k(t_flat, idx32)


def kernel(t, idx):
    return _sc_gather(t.reshape(R * V), idx.astype(jnp.int32))
